# baseline (speedup 1.0000x reference)
import functools

import jax
import jax.numpy as jnp
from jax import lax
from jax.experimental import pallas as pl
from jax.experimental.pallas import tpu as pltpu
from jax.experimental.pallas import tpu_sc as plsc

_NC = 2
_NS = 16
_LANES = 16
_RC = 16


def _sc_add(B, T, C, x_hbm, pe_hbm, out_hbm,
            in0, in1, out0, out1, pe_v, l0, l1, s0, s1):
    nw = _NC * _NS
    wid = lax.axis_index("s") * _NC + lax.axis_index("c")
    rows_per_w = T // nw
    t0 = wid * rows_per_w
    n_chunks = rows_per_w // _RC
    n_j = n_chunks * B

    ins = (in0, in1)
    outs = (out0, out1)
    lsems = (l0, l1)
    ssems = (s0, s1)

    def row0(j):
        return t0 + (j // B) * _RC

    def load(j):
        return pltpu.make_async_copy(
            x_hbm.at[j % B, pl.ds(row0(j), _RC)], ins[j % 2], lsems[j % 2])

    def store(j):
        return pltpu.make_async_copy(
            outs[j % 2], out_hbm.at[j % B, pl.ds(row0(j), _RC)], ssems[j % 2])

    load(0).start()
    load(1).start()
    for j in range(n_j):
        p = j % 2
        if j % B == 0:
            pltpu.sync_copy(pe_hbm.at[pl.ds(row0(j), _RC)], pe_v)
        load(j).wait()
        if j >= 2:
            store(j - 2).wait()
        xb, ob = ins[p], outs[p]

        unr = 8
        vecs_per_row = C // (_LANES * unr)
        def col_body(i, _, xb=xb, ob=ob):
            r = i // vecs_per_row
            base = (i % vecs_per_row) * (_LANES * unr)
            for k in range(unr):
                s = base + k * _LANES
                ob[r, pl.ds(s, _LANES)] = (xb[r, pl.ds(s, _LANES)]
                                           + pe_v[r, pl.ds(s, _LANES)])
            return _
        lax.fori_loop(0, _RC * vecs_per_row, col_body, 0)

        store(j).start()
        if j + 2 < n_j:
            load(j + 2).start()
    store(n_j - 2).wait()
    store(n_j - 1).wait()


def kernel(x, pos_emb):
    B, T, C = x.shape
    mesh = plsc.VectorSubcoreMesh(core_axis_name="c", subcore_axis_name="s")
    f32 = jnp.float32
    run = pl.kernel(
        functools.partial(_sc_add, B, T, C),
        out_type=jax.ShapeDtypeStruct((B, T, C), f32),
        mesh=mesh,
        scratch_types=[
            pltpu.VMEM((_RC, C), f32),
            pltpu.VMEM((_RC, C), f32),
            pltpu.VMEM((_RC, C), f32),
            pltpu.VMEM((_RC, C), f32),
            pltpu.VMEM((_RC, C), f32),
            pltpu.SemaphoreType.DMA,
            pltpu.SemaphoreType.DMA,
            pltpu.SemaphoreType.DMA,
            pltpu.SemaphoreType.DMA,
        ],
    )
    return run(x, pos_emb)
